# baseline (speedup 1.0000x reference)
import functools

import jax
import jax.numpy as jnp
from jax import lax
from jax.experimental import pallas as pl
from jax.experimental.pallas import tpu as pltpu
from jax.experimental.pallas import tpu_sc as plsc

D = 64
C = 128
NBUF = 4
L16 = 16


def _fuse_tables(Wt, Bt, V):
    info = plsc.get_sparse_core_info()
    NC, NS = info.num_cores, info.num_subcores
    NW = NC * NS
    nfull = V // C
    rem = nfull % NW
    base = nfull // NW

    mesh = plsc.VectorSubcoreMesh(core_axis_name="c", subcore_axis_name="s")

    @functools.partial(
        pl.kernel,
        mesh=mesh,
        out_type=jax.ShapeDtypeStruct((V, 2 * D), jnp.float32),
        scratch_types=[
            pltpu.VMEM((2, D, C), jnp.float32),
            pltpu.VMEM((2, D, C), jnp.float32),
            pltpu.VMEM((2, C, 2 * D), jnp.float32),
            [pltpu.SemaphoreType.DMA] * 2,
            [pltpu.SemaphoreType.DMA] * 2,
        ],
        compiler_params=pltpu.CompilerParams(needs_layout_passes=False),
    )
    def fuse(wt_hbm, bt_hbm, fused_hbm, stw, stb, blk, ssems, wsems):
        wid = lax.axis_index("s") * NC + lax.axis_index("c")
        nb = base + jnp.where(wid < rem, 1, 0)

        def c0_of(k):
            return pl.multiple_of((k * NW + wid) * C, C)

        def stage_desc(k, s):
            c0 = c0_of(k)
            return (
                pltpu.make_async_copy(wt_hbm.at[:, pl.ds(c0, C)], stw.at[s], ssems[s]),
                pltpu.make_async_copy(bt_hbm.at[:, pl.ds(c0, C)], stb.at[s], ssems[s]),
            )

        def write_desc(k, s):
            c0 = c0_of(k)
            return (
                pltpu.make_async_copy(blk.at[s], fused_hbm.at[pl.ds(c0, C)], wsems[s]),
            )

        def start(ds):
            for d in ds:
                d.start()

        def wait(ds):
            for d in ds:
                d.wait()

        rows = lax.iota(jnp.int32, L16)

        def transpose(s):
            def vbody(v, carry):
                col = jnp.zeros((L16,), jnp.int32) + v
                for g in range(D // L16):
                    blk[s, v, pl.ds(g * L16, L16)] = plsc.load_gather(
                        stw.at[s], [rows + g * L16, col])
                    blk[s, v, pl.ds(D + g * L16, L16)] = plsc.load_gather(
                        stb.at[s], [rows + g * L16, col])
                return carry

            lax.fori_loop(0, C, vbody, 0)

        def step(k, s, first):
            wait(stage_desc(k, s))

            @pl.when(k + 1 < nb)
            def _():
                start(stage_desc(k + 1, 1 - s))

            if not first:
                wait(write_desc(k - 2, s))
            transpose(s)
            start(write_desc(k, s))

        start(stage_desc(0, 0))
        step(0, 0, True)
        step(1, 1, True)

        def body(jj, carry):
            for s in range(2):
                step(jj * 2 + s, s, False)
            return carry

        lax.fori_loop(1, base // 2, body, 0)
        for k in range(2 * (base // 2), base):
            step(k, k % 2, False)

        @pl.when(nb > base)
        def _():
            step(base, base % 2, False)

        @pl.when(nb % 2 == 0)
        def _():
            wait(write_desc(nb - 2, 0))
            wait(write_desc(nb - 1, 1))

        @pl.when(nb % 2 == 1)
        def _():
            wait(write_desc(nb - 2, 1))
            wait(write_desc(nb - 1, 0))

    return fuse(Wt, Bt)


def kernel(ids, W_word, W_bert):
    B, L = ids.shape
    N = B * L
    V, _ = W_word.shape
    info = plsc.get_sparse_core_info()
    NC, NS = info.num_cores, info.num_subcores
    NW = NC * NS
    nchunk_total = N // C
    nchunk = nchunk_total // NW
    assert nchunk * NW * C == N and nchunk % NBUF == 0 and nchunk >= 2 * NBUF

    ids2 = ids.reshape(nchunk_total, C).astype(jnp.int32)

    Vm = (V // C) * C
    WC = _fuse_tables(jnp.transpose(W_word), jnp.transpose(W_bert), V)
    tail = jnp.concatenate([W_word[Vm:], W_bert[Vm:]], axis=1)
    WC = lax.dynamic_update_slice(WC, tail, (Vm, 0))

    mesh = plsc.VectorSubcoreMesh(core_axis_name="c", subcore_axis_name="s")

    @functools.partial(
        pl.kernel,
        mesh=mesh,
        out_type=jax.ShapeDtypeStruct((nchunk_total, C, 2 * D), jnp.float32),
        scratch_types=[
            pltpu.VMEM((nchunk, C), jnp.int32),
            pltpu.VMEM((NBUF, C, 2 * D), jnp.float32),
            [pltpu.SemaphoreType.DMA] * NBUF,
            [pltpu.SemaphoreType.DMA] * NBUF,
        ],
        compiler_params=pltpu.CompilerParams(use_tc_tiling_on_sc=False),
    )
    def run(ids_hbm, wc_hbm, out_hbm, idx_v, cbuf, gsems, wsems):
        wid = lax.axis_index("s") * NC + lax.axis_index("c")
        crow = wid * nchunk
        pltpu.sync_copy(ids_hbm.at[pl.ds(crow, nchunk)], idx_v)

        def g_desc(j, s):
            return (
                pltpu.make_async_copy(wc_hbm.at[idx_v.at[j]], cbuf.at[s], gsems[s]),
            )

        def w_desc(j, s):
            return (
                pltpu.make_async_copy(cbuf.at[s], out_hbm.at[crow + j], wsems[s]),
            )

        def start(ds):
            for d in ds:
                d.start()

        def wait(ds):
            for d in ds:
                d.wait()

        for j in range(NBUF - 1):
            start(g_desc(j, j))
        for j in range(NBUF):
            wait(g_desc(j, j))
            start(w_desc(j, j))
            if j == 0:
                start(g_desc(NBUF - 1, NBUF - 1))
            else:
                wait(w_desc(j - 1, j - 1))
                start(g_desc(j + NBUF - 1, (j - 1) % NBUF))

        def body(jj, carry):
            for s in range(NBUF):
                j = jj * NBUF + s
                s2 = (s + NBUF - 1) % NBUF
                wait(g_desc(j, s))
                start(w_desc(j, s))
                wait(w_desc(j - 1, s2))
                start(g_desc(j + NBUF - 1, s2))
            return carry

        lax.fori_loop(1, nchunk // NBUF - 1, body, 0)

        for j in range(nchunk - NBUF, nchunk):
            s = j % NBUF
            s2 = (s + NBUF - 1) % NBUF
            wait(g_desc(j, s))
            start(w_desc(j, s))
            if j == nchunk - NBUF:
                wait(w_desc(j - 1, s2))
                start(g_desc(j + NBUF - 1, s2))
        for j in range(nchunk - NBUF, nchunk):
            wait(w_desc(j, j % NBUF))

    out = run(ids2, WC)
    return out.reshape(B, L, 2 * D)


# speedup vs baseline: 1.7538x; 1.7538x over previous
import functools

import jax
import jax.numpy as jnp
from jax import lax
from jax.experimental import pallas as pl
from jax.experimental.pallas import tpu as pltpu
from jax.experimental.pallas import tpu_sc as plsc

D = 64
C = 128
NBUF = 4
L16 = 16


def _fuse_tables(Wt, Bt, V):
    info = plsc.get_sparse_core_info()
    NC, NS = info.num_cores, info.num_subcores
    NW = NC * NS
    nfull = V // C
    rem = nfull % NW
    base = nfull // NW

    mesh = plsc.VectorSubcoreMesh(core_axis_name="c", subcore_axis_name="s")

    @functools.partial(
        pl.kernel,
        mesh=mesh,
        out_type=jax.ShapeDtypeStruct((V, 2 * D), jnp.float32),
        scratch_types=[
            pltpu.VMEM((2, D, C), jnp.float32),
            pltpu.VMEM((2, D, C), jnp.float32),
            pltpu.VMEM((2, C, 2 * D), jnp.float32),
            [pltpu.SemaphoreType.DMA] * 2,
            [pltpu.SemaphoreType.DMA] * 2,
        ],
        compiler_params=pltpu.CompilerParams(needs_layout_passes=False),
    )
    def fuse(wt_hbm, bt_hbm, fused_hbm, stw, stb, blk, ssems, wsems):
        wid = lax.axis_index("s") * NC + lax.axis_index("c")
        nb = base + jnp.where(wid < rem, 1, 0)

        def c0_of(k):
            return pl.multiple_of((k * NW + wid) * C, C)

        def stage_desc(k, s):
            c0 = c0_of(k)
            return (
                pltpu.make_async_copy(wt_hbm.at[:, pl.ds(c0, C)], stw.at[s], ssems[s]),
                pltpu.make_async_copy(bt_hbm.at[:, pl.ds(c0, C)], stb.at[s], ssems[s]),
            )

        def write_desc(k, s):
            c0 = c0_of(k)
            return (
                pltpu.make_async_copy(blk.at[s], fused_hbm.at[pl.ds(c0, C)], wsems[s]),
            )

        def start(ds):
            for d in ds:
                d.start()

        def wait(ds):
            for d in ds:
                d.wait()

        rows = lax.iota(jnp.int32, L16)
        rg = [rows + g * L16 for g in range(D // L16)]

        def transpose(s):
            @plsc.parallel_loop(0, C, step=1, unroll=8)
            def vbody(v):
                col = jnp.zeros((L16,), jnp.int32) + v
                for g in range(D // L16):
                    blk[s, v, pl.ds(g * L16, L16)] = plsc.load_gather(
                        stw.at[s], [rg[g], col])
                    blk[s, v, pl.ds(D + g * L16, L16)] = plsc.load_gather(
                        stb.at[s], [rg[g], col])

        def step(k, s, first):
            wait(stage_desc(k, s))

            @pl.when(k + 1 < nb)
            def _():
                start(stage_desc(k + 1, 1 - s))

            if not first:
                wait(write_desc(k - 2, s))
            transpose(s)
            start(write_desc(k, s))

        start(stage_desc(0, 0))
        step(0, 0, True)
        step(1, 1, True)

        def body(jj, carry):
            for s in range(2):
                step(jj * 2 + s, s, False)
            return carry

        lax.fori_loop(1, base // 2, body, 0)
        for k in range(2 * (base // 2), base):
            step(k, k % 2, False)

        @pl.when(nb > base)
        def _():
            step(base, base % 2, False)

        @pl.when(nb % 2 == 0)
        def _():
            wait(write_desc(nb - 2, 0))
            wait(write_desc(nb - 1, 1))

        @pl.when(nb % 2 == 1)
        def _():
            wait(write_desc(nb - 2, 1))
            wait(write_desc(nb - 1, 0))

    return fuse(Wt, Bt)


def kernel(ids, W_word, W_bert):
    B, L = ids.shape
    N = B * L
    V, _ = W_word.shape
    info = plsc.get_sparse_core_info()
    NC, NS = info.num_cores, info.num_subcores
    NW = NC * NS
    nchunk_total = N // C
    nchunk = nchunk_total // NW
    assert nchunk * NW * C == N and nchunk % NBUF == 0 and nchunk >= 2 * NBUF

    ids2 = ids.reshape(nchunk_total, C).astype(jnp.int32)

    Vm = (V // C) * C
    WC = _fuse_tables(jnp.transpose(W_word), jnp.transpose(W_bert), V)
    tail = jnp.concatenate([W_word[Vm:], W_bert[Vm:]], axis=1)
    WC = lax.dynamic_update_slice(WC, tail, (Vm, 0))

    mesh = plsc.VectorSubcoreMesh(core_axis_name="c", subcore_axis_name="s")

    @functools.partial(
        pl.kernel,
        mesh=mesh,
        out_type=jax.ShapeDtypeStruct((nchunk_total, C, 2 * D), jnp.float32),
        scratch_types=[
            pltpu.VMEM((nchunk, C), jnp.int32),
            pltpu.VMEM((NBUF, C, 2 * D), jnp.float32),
            [pltpu.SemaphoreType.DMA] * NBUF,
            [pltpu.SemaphoreType.DMA] * NBUF,
        ],
        compiler_params=pltpu.CompilerParams(use_tc_tiling_on_sc=False),
    )
    def run(ids_hbm, wc_hbm, out_hbm, idx_v, cbuf, gsems, wsems):
        wid = lax.axis_index("s") * NC + lax.axis_index("c")
        crow = wid * nchunk
        pltpu.sync_copy(ids_hbm.at[pl.ds(crow, nchunk)], idx_v)

        def g_desc(j, s):
            return (
                pltpu.make_async_copy(wc_hbm.at[idx_v.at[j]], cbuf.at[s], gsems[s]),
            )

        def w_desc(j, s):
            return (
                pltpu.make_async_copy(cbuf.at[s], out_hbm.at[crow + j], wsems[s]),
            )

        def start(ds):
            for d in ds:
                d.start()

        def wait(ds):
            for d in ds:
                d.wait()

        for j in range(NBUF - 1):
            start(g_desc(j, j))
        for j in range(NBUF):
            wait(g_desc(j, j))
            start(w_desc(j, j))
            if j == 0:
                start(g_desc(NBUF - 1, NBUF - 1))
            else:
                wait(w_desc(j - 1, j - 1))
                start(g_desc(j + NBUF - 1, (j - 1) % NBUF))

        def body(jj, carry):
            for s in range(NBUF):
                j = jj * NBUF + s
                s2 = (s + NBUF - 1) % NBUF
                wait(g_desc(j, s))
                start(w_desc(j, s))
                wait(w_desc(j - 1, s2))
                start(g_desc(j + NBUF - 1, s2))
            return carry

        lax.fori_loop(1, nchunk // NBUF - 1, body, 0)

        for j in range(nchunk - NBUF, nchunk):
            s = j % NBUF
            s2 = (s + NBUF - 1) % NBUF
            wait(g_desc(j, s))
            start(w_desc(j, s))
            if j == nchunk - NBUF:
                wait(w_desc(j - 1, s2))
                start(g_desc(j + NBUF - 1, s2))
        for j in range(nchunk - NBUF, nchunk):
            wait(w_desc(j, j % NBUF))

    out = run(ids2, WC)
    return out.reshape(B, L, 2 * D)


# speedup vs baseline: 2.7560x; 1.5714x over previous
import functools

import jax
import jax.numpy as jnp
from jax import lax
from jax.experimental import pallas as pl
from jax.experimental.pallas import tpu as pltpu
from jax.experimental.pallas import tpu_sc as plsc

D = 64
C = 128
NBUF = 4


def kernel(ids, W_word, W_bert):
    B, L = ids.shape
    N = B * L
    info = plsc.get_sparse_core_info()
    NC, NS = info.num_cores, info.num_subcores
    NW = NC * NS
    nchunk_total = N // C
    nchunk = nchunk_total // NW
    assert nchunk * NW * C == N and nchunk % NBUF == 0 and nchunk >= 2 * NBUF

    ids2 = ids.reshape(nchunk_total, C).astype(jnp.int32)
    WC = jnp.concatenate([W_word, W_bert], axis=1)

    mesh = plsc.VectorSubcoreMesh(core_axis_name="c", subcore_axis_name="s")

    @functools.partial(
        pl.kernel,
        mesh=mesh,
        out_type=jax.ShapeDtypeStruct((nchunk_total, C, 2 * D), jnp.float32),
        scratch_types=[
            pltpu.VMEM((nchunk, C), jnp.int32),
            pltpu.VMEM((NBUF, C, 2 * D), jnp.float32),
            [pltpu.SemaphoreType.DMA] * NBUF,
            [pltpu.SemaphoreType.DMA] * NBUF,
        ],
        compiler_params=pltpu.CompilerParams(use_tc_tiling_on_sc=False),
    )
    def run(ids_hbm, wc_hbm, out_hbm, idx_v, cbuf, gsems, wsems):
        wid = lax.axis_index("s") * NC + lax.axis_index("c")
        crow = wid * nchunk
        pltpu.sync_copy(ids_hbm.at[pl.ds(crow, nchunk)], idx_v)

        def g_desc(j, s):
            return (
                pltpu.make_async_copy(wc_hbm.at[idx_v.at[j]], cbuf.at[s], gsems[s]),
            )

        def w_desc(j, s):
            return (
                pltpu.make_async_copy(cbuf.at[s], out_hbm.at[crow + j], wsems[s]),
            )

        def start(ds):
            for d in ds:
                d.start()

        def wait(ds):
            for d in ds:
                d.wait()

        for j in range(NBUF - 1):
            start(g_desc(j, j))
        for j in range(NBUF):
            wait(g_desc(j, j))
            start(w_desc(j, j))
            if j == 0:
                start(g_desc(NBUF - 1, NBUF - 1))
            else:
                wait(w_desc(j - 1, j - 1))
                start(g_desc(j + NBUF - 1, (j - 1) % NBUF))

        def body(jj, carry):
            for s in range(NBUF):
                j = jj * NBUF + s
                s2 = (s + NBUF - 1) % NBUF
                wait(g_desc(j, s))
                start(w_desc(j, s))
                wait(w_desc(j - 1, s2))
                start(g_desc(j + NBUF - 1, s2))
            return carry

        lax.fori_loop(1, nchunk // NBUF - 1, body, 0)

        for j in range(nchunk - NBUF, nchunk):
            s = j % NBUF
            s2 = (s + NBUF - 1) % NBUF
            wait(g_desc(j, s))
            start(w_desc(j, s))
            if j == nchunk - NBUF:
                wait(w_desc(j - 1, s2))
                start(g_desc(j + NBUF - 1, s2))
        for j in range(nchunk - NBUF, nchunk):
            wait(w_desc(j, j % NBUF))

    out = run(ids2, WC)
    return out.reshape(B, L, 2 * D)
